# Initial kernel scaffold; baseline (speedup 1.0000x reference)
import jax
import jax.numpy as jnp
from jax.experimental import pallas as pl


def kernel(unknown, known, unknow_feats, known_feats, W1, g1, b1, W2, g2, b2):
    raise NotImplementedError("write your pallas kernel here")


# speedup vs baseline: 14.1915x; 14.1915x over previous
import functools

import jax
import jax.numpy as jnp
from jax import lax
from jax.experimental import pallas as pl
from jax.experimental.pallas import tpu as pltpu
from jax.experimental.pallas import tpu_sc as plsc

B, N, M = 8, 4096, 1024
C1, C2 = 128, 256
CMID, COUT = 256, 256
TN = 1024
NT = N // TN
BN_COUNT = float(B * N)
H = B // 2

NW = 32
CH = (H * N) // NW
G = 128
NSUB = CH // G


def _stage1_body(u_ref, kn_ref, gi_ref, gw_ref, *, half):
    u = u_ref[0]
    kn = kn_ref[0]
    k2 = jnp.sum(kn * kn, axis=1, keepdims=True)
    ub = u.astype(jnp.bfloat16)
    knb = kn.astype(jnp.bfloat16)
    ku = jax.lax.dot_general(knb, ub, (((1,), (1,)), ((), ())),
                             preferred_element_type=jnp.float32)
    usq = u * u
    u2 = jax.lax.dot_general(jnp.ones((1, 3), jnp.float32), usq,
                             (((1,), (1,)), ((), ())),
                             precision=jax.lax.Precision.HIGHEST,
                             preferred_element_type=jnp.float32)
    d = (u2 + k2) - 2.0 * ku

    iota = jax.lax.broadcasted_iota(jnp.int32, (M, TN), 0)
    vs, ids = [], []
    for k in range(3):
        v = jnp.min(d, axis=0, keepdims=True)
        hit = d == v
        idx = jnp.min(jnp.where(hit, iota, M), axis=0, keepdims=True)
        vs.append(v)
        ids.append(idx)
        if k < 2:
            d = jnp.where(iota == idx, jnp.float32(jnp.inf), d)

    recips = [1.0 / (v + 1e-8) for v in vs]
    norm = recips[0] + recips[1] + recips[2]
    w = [r / norm for r in recips]

    boff = (pl.program_id(0) + half * H) * M
    zi = jnp.zeros((5, TN), jnp.int32)
    gi_ref[0] = jnp.concatenate([ids[0] + boff, ids[1] + boff,
                                 ids[2] + boff, zi], axis=0)
    zf = jnp.zeros((5, TN), jnp.float32)
    w8 = jnp.concatenate(w + [zf], axis=0)
    rows = jax.lax.broadcasted_iota(jnp.int32, (TN, TN), 0)
    cols = jax.lax.broadcasted_iota(jnp.int32, (TN, TN), 1)
    eye = (rows == cols).astype(jnp.float32)
    gw_ref[...] = jax.lax.dot_general(eye, w8, (((1,), (1,)), ((), ())),
                                      precision=jax.lax.Precision.HIGHEST,
                                      preferred_element_type=jnp.float32)


def _sc_gather(gi_hbm, tab_hbm, out_hbm, i0, i1, i2, b0, b1, b2, sem):
    cid = lax.axis_index("c")
    sid = lax.axis_index("s")
    wid = sid * 2 + cid
    base = wid * CH
    b = base // N
    nb = base - b * N

    def sub(s, carry):
        n0 = nb + s * G
        pltpu.sync_copy(gi_hbm.at[b, 0, pl.ds(n0, G)], i0)
        pltpu.sync_copy(gi_hbm.at[b, 1, pl.ds(n0, G)], i1)
        pltpu.sync_copy(gi_hbm.at[b, 2, pl.ds(n0, G)], i2)
        c0 = pltpu.async_copy(tab_hbm.at[i0], b0, sem)
        c1 = pltpu.async_copy(tab_hbm.at[i1], b1, sem)
        c2 = pltpu.async_copy(tab_hbm.at[i2], b2, sem)
        c0.wait()
        c1.wait()
        c2.wait()
        row = base + s * G
        pltpu.sync_copy(b0, out_hbm.at[0, pl.ds(row, G)])
        pltpu.sync_copy(b1, out_hbm.at[1, pl.ds(row, G)])
        pltpu.sync_copy(b2, out_hbm.at[2, pl.ds(row, G)])
        return carry

    lax.fori_loop(0, NSUB, sub, 0)


_sc_gather_call = functools.partial(
    pl.kernel,
    mesh=plsc.VectorSubcoreMesh(core_axis_name="c", subcore_axis_name="s"),
    out_type=jax.ShapeDtypeStruct((3, H * N, C2), jnp.float32),
    scratch_types=[
        pltpu.VMEM((G,), jnp.int32),
        pltpu.VMEM((G,), jnp.int32),
        pltpu.VMEM((G,), jnp.int32),
        pltpu.VMEM((G, C2), jnp.float32),
        pltpu.VMEM((G, C2), jnp.float32),
        pltpu.VMEM((G, C2), jnp.float32),
        pltpu.SemaphoreType.DMA,
    ],
)(_sc_gather)


def _stage1b_body(r_ref, wt_ref, uf_ref, w1_ref, y1_ref, s_ref, q_ref):
    w1a = w1_ref[:, :C2].astype(jnp.bfloat16)
    w1b = w1_ref[:, C2:].astype(jnp.bfloat16)
    ufb = uf_ref[0].astype(jnp.bfloat16)
    interp = ((wt_ref[:, 0:1] * r_ref[0] + wt_ref[:, 1:2] * r_ref[1])
              + wt_ref[:, 2:3] * r_ref[2])
    y1 = (jax.lax.dot_general(w1a, interp.astype(jnp.bfloat16),
                              (((1,), (1,)), ((), ())),
                              preferred_element_type=jnp.float32)
          + jax.lax.dot_general(w1b, ufb, (((1,), (0,)), ((), ())),
                                preferred_element_type=jnp.float32))
    y1_ref[0] = y1

    ps = jnp.sum(y1, axis=1, keepdims=True)
    pq = jnp.sum(y1 * y1, axis=1, keepdims=True)

    @pl.when(jnp.logical_and(pl.program_id(0) == 0, pl.program_id(1) == 0))
    def _init():
        s_ref[...] = ps
        q_ref[...] = pq

    @pl.when(jnp.logical_or(pl.program_id(0) != 0, pl.program_id(1) != 0))
    def _acc():
        s_ref[...] += ps
        q_ref[...] += pq


def _stage2_body(y1_ref, sa_ref, sb_ref, qa_ref, qb_ref, g_ref, b_ref,
                 w2_ref, y2_ref, s2_ref, q2_ref):
    mean = (sa_ref[...] + sb_ref[...]) * (1.0 / BN_COUNT)
    var = (qa_ref[...] + qb_ref[...]) * (1.0 / BN_COUNT) - mean * mean
    scale = g_ref[...] * jax.lax.rsqrt(var + 1e-5)
    shift = b_ref[...] - mean * scale
    h = jnp.maximum(y1_ref[0] * scale + shift, 0.0)
    y2 = jax.lax.dot_general(w2_ref[...].astype(jnp.bfloat16),
                             h.astype(jnp.bfloat16),
                             (((1,), (0,)), ((), ())),
                             preferred_element_type=jnp.float32)
    y2_ref[0] = y2

    ps = jnp.sum(y2, axis=1, keepdims=True)
    pq = jnp.sum(y2 * y2, axis=1, keepdims=True)

    @pl.when(jnp.logical_and(pl.program_id(0) == 0, pl.program_id(1) == 0))
    def _init():
        s2_ref[...] = ps
        q2_ref[...] = pq

    @pl.when(jnp.logical_or(pl.program_id(0) != 0, pl.program_id(1) != 0))
    def _acc():
        s2_ref[...] += ps
        q2_ref[...] += pq


def _stage3_body(y2_ref, sa_ref, sb_ref, qa_ref, qb_ref, g_ref, b_ref,
                 out_ref):
    mean = (sa_ref[...] + sb_ref[...]) * (1.0 / BN_COUNT)
    var = (qa_ref[...] + qb_ref[...]) * (1.0 / BN_COUNT) - mean * mean
    scale = g_ref[...] * jax.lax.rsqrt(var + 1e-5)
    shift = b_ref[...] - mean * scale
    out_ref[0] = jnp.maximum(y2_ref[0] * scale + shift, 0.0)


_STAT_SPEC = pl.BlockSpec((CMID, 1), lambda b, t: (0, 0))


@jax.jit
def kernel(unknown, known, unknow_feats, known_feats, W1, g1, b1, W2, g2, b2):
    f32 = jnp.float32
    table = known_feats.transpose(0, 2, 1).reshape(B * M, C2)

    gi, gw, rows3, uf_h = [], [], [], []
    for h in range(2):
        sl = slice(h * H, (h + 1) * H)
        gih, gwh = pl.pallas_call(
            functools.partial(_stage1_body, half=h),
            grid=(H, NT),
            in_specs=[
                pl.BlockSpec((1, TN, 3), lambda b, t: (b, t, 0)),
                pl.BlockSpec((1, M, 3), lambda b, t: (b, 0, 0)),
            ],
            out_specs=[
                pl.BlockSpec((1, 8, TN), lambda b, t: (b, 0, t)),
                pl.BlockSpec((TN, 8), lambda b, t: (b * NT + t, 0)),
            ],
            out_shape=[
                jax.ShapeDtypeStruct((H, 8, N), jnp.int32),
                jax.ShapeDtypeStruct((H * N, 8), f32),
            ],
        )(unknown[sl], known[sl])
        gi.append(gih)
        gw.append(gwh)
        uf_h.append(unknow_feats[sl])

    for h in range(2):
        rows3.append(_sc_gather_call(gi[h], table))

    y1_h, s1_h, q1_h = [], [], []
    for h in range(2):
        y1h, s1h, q1h = pl.pallas_call(
            _stage1b_body,
            grid=(H, NT),
            in_specs=[
                pl.BlockSpec((3, TN, C2), lambda b, t: (0, b * NT + t, 0)),
                pl.BlockSpec((TN, 8), lambda b, t: (b * NT + t, 0)),
                pl.BlockSpec((1, C1, TN), lambda b, t: (b, 0, t)),
                pl.BlockSpec((CMID, C1 + C2), lambda b, t: (0, 0)),
            ],
            out_specs=[
                pl.BlockSpec((1, CMID, TN), lambda b, t: (b, 0, t)),
                _STAT_SPEC,
                _STAT_SPEC,
            ],
            out_shape=[
                jax.ShapeDtypeStruct((H, CMID, N), f32),
                jax.ShapeDtypeStruct((CMID, 1), f32),
                jax.ShapeDtypeStruct((CMID, 1), f32),
            ],
        )(rows3[h], gw[h], uf_h[h], W1)
        y1_h.append(y1h)
        s1_h.append(s1h)
        q1_h.append(q1h)

    y2_h, s2_h, q2_h = [], [], []
    for h in range(2):
        y2h, s2h, q2h = pl.pallas_call(
            _stage2_body,
            grid=(H, NT),
            in_specs=[
                pl.BlockSpec((1, CMID, TN), lambda b, t: (b, 0, t)),
                _STAT_SPEC, _STAT_SPEC, _STAT_SPEC, _STAT_SPEC,
                _STAT_SPEC, _STAT_SPEC,
                pl.BlockSpec((COUT, CMID), lambda b, t: (0, 0)),
            ],
            out_specs=[
                pl.BlockSpec((1, COUT, TN), lambda b, t: (b, 0, t)),
                _STAT_SPEC,
                _STAT_SPEC,
            ],
            out_shape=[
                jax.ShapeDtypeStruct((H, COUT, N), f32),
                jax.ShapeDtypeStruct((COUT, 1), f32),
                jax.ShapeDtypeStruct((COUT, 1), f32),
            ],
        )(y1_h[h], s1_h[0], s1_h[1], q1_h[0], q1_h[1],
          g1.reshape(CMID, 1), b1.reshape(CMID, 1), W2)
        y2_h.append(y2h)
        s2_h.append(s2h)
        q2_h.append(q2h)

    out_h = []
    for h in range(2):
        out_h.append(pl.pallas_call(
            _stage3_body,
            grid=(H, NT),
            in_specs=[
                pl.BlockSpec((1, COUT, TN), lambda b, t: (b, 0, t)),
                _STAT_SPEC, _STAT_SPEC, _STAT_SPEC, _STAT_SPEC,
                _STAT_SPEC, _STAT_SPEC,
            ],
            out_specs=pl.BlockSpec((1, COUT, TN), lambda b, t: (b, 0, t)),
            out_shape=jax.ShapeDtypeStruct((H, COUT, N), f32),
        )(y2_h[h], s2_h[0], s2_h[1], q2_h[0], q2_h[1],
          g2.reshape(COUT, 1), b2.reshape(COUT, 1)))

    return jnp.concatenate(out_h, axis=0)
